# baseline (speedup 1.0000x reference)
import functools

import jax
import jax.numpy as jnp
from jax import lax
from jax.experimental import pallas as pl
from jax.experimental.pallas import tpu as pltpu
from jax.experimental.pallas import tpu_sc as plsc

VOCAB_ = 1000000
D = 64
NUM_CORES = 2
NUM_SUBCORES = 16
NW = NUM_CORES * NUM_SUBCORES
LANES = 16

FULL_UNITS = VOCAB_ // 128
TAIL_PAIRS = (VOCAB_ - FULL_UNITS * 128) // 2
T_ROUNDS = (FULL_UNITS + NW - 1) // NW // 2 + 2

J = 200
I_ = 4096
JG = J // 8
IB = I_ // 128
SB_PER_W = (JG * IB) // NW


def _mesh():
    return plsc.VectorSubcoreMesh(core_axis_name="c", subcore_axis_name="s")


def _wid():
    return lax.axis_index("s") * NUM_CORES + lax.axis_index("c")


@functools.partial(
    pl.kernel,
    out_type=jax.ShapeDtypeStruct((VOCAB_ // 2, 128), jnp.float32),
    mesh=_mesh(),
    scratch_types=[
        pltpu.VMEM((2, D, 128), jnp.float32),
        pltpu.VMEM((2, D, 128), jnp.float32),
        pltpu.SemaphoreType.DMA((2,)),
        pltpu.SemaphoreType.DMA((2,)),
    ],
    compiler_params=pltpu.CompilerParams(use_tc_tiling_on_sc=True,
                                         needs_layout_passes=False),
)
def _transpose_call(tableT, tail_tp, tp, blk, pck, isem, osem):
    w = _wid()
    iot = lax.iota(jnp.int32, LANES)

    def unit_of(t):
        return w + NW * t

    def in_copy(t, b):
        c0 = pl.multiple_of(unit_of(t) * 128, 128)
        return pltpu.make_async_copy(
            tableT.at[:, pl.ds(c0, 128)], blk.at[b], isem.at[b]
        )

    def out_copy(t, b):
        r0 = pl.multiple_of(unit_of(t) * D, 8)
        return pltpu.make_async_copy(
            pck.at[b], tp.at[pl.ds(r0, D), :], osem.at[b]
        )

    def do_unit(b):
        def krow(k, carry):
            for half in range(2):
                col = jnp.full((LANES,), 2 * k + half, jnp.int32)
                for g in range(D // LANES):
                    vals = plsc.load_gather(blk.at[b], [iot + 16 * g, col])
                    pck[b, k, pl.ds(64 * half + 16 * g, LANES)] = vals
            return carry

        lax.fori_loop(0, 64, krow, 0)

    for b in range(2):
        @pl.when(unit_of(b) < FULL_UNITS)
        def _():
            in_copy(b, b).start()

    def rounds(tt, carry):
        for b in range(2):
            t = tt * 2 + b

            @pl.when((t >= 2) & (unit_of(t - 2) < FULL_UNITS))
            def _():
                out_copy(t - 2, b).wait()

            @pl.when(unit_of(t) < FULL_UNITS)
            def _():
                in_copy(t, b).wait()
                do_unit(b)
                out_copy(t, b).start()

                @pl.when(unit_of(t + 2) < FULL_UNITS)
                def _():
                    in_copy(t + 2, b).start()
        return carry

    lax.fori_loop(0, T_ROUNDS, rounds, 0)

    @pl.when(w == 0)
    def _():
        pltpu.sync_copy(tail_tp, tp.at[pl.ds(FULL_UNITS * D, TAIL_PAIRS), :])


@functools.partial(
    pl.kernel,
    out_type=jax.ShapeDtypeStruct((J, D, I_), jnp.float32),
    mesh=_mesh(),
    scratch_types=[
        pltpu.VMEM((8, 128), jnp.int32),
        pltpu.VMEM((8, 128), jnp.int32),
        pltpu.VMEM((8, 128), jnp.int32),
        pltpu.VMEM((2, 128, 128), jnp.float32),
        pltpu.VMEM((2, D, 128), jnp.float32),
        pltpu.SemaphoreType.DMA,
        pltpu.SemaphoreType.DMA((2,)),
        pltpu.SemaphoreType.DMA((2,)),
    ],
    compiler_params=pltpu.CompilerParams(use_tc_tiling_on_sc=True,
                                         needs_layout_passes=False),
)
def _gather_call(xt, tp, out3, xv, pv, qv, gat, slab, xsem, gsem, osem):
    w = _wid()
    iot = lax.iota(jnp.int32, LANES)

    def idx_copy(sb):
        jg = sb // IB
        ib = lax.rem(sb, IB)
        return pltpu.make_async_copy(
            xt.at[pl.ds(pl.multiple_of(jg * 8, 8), 8),
                  pl.ds(pl.multiple_of(ib * 128, 128), 128)],
            xv, xsem)

    def gather_copy(r, b):
        return pltpu.make_async_copy(tp.at[pv.at[r]], gat.at[b], gsem.at[b])

    def out_copy(sb, r, b):
        jg = sb // IB
        ib = lax.rem(sb, IB)
        return pltpu.make_async_copy(
            slab.at[b],
            out3.at[jg * 8 + r, :, pl.ds(pl.multiple_of(ib * 128, 128), 128)],
            osem.at[b])

    def compute_pidx():
        def row(r, carry):
            for g in range(128 // LANES):
                v = xv[r, pl.ds(16 * g, LANES)]
                pv[r, pl.ds(16 * g, LANES)] = lax.shift_right_logical(v, 1)
                qv[r, pl.ds(16 * g, LANES)] = lax.shift_left(
                    lax.bitwise_and(v, 1), 6)
            return carry
        lax.fori_loop(0, 8, row, 0)

    def select_transpose(r, b):
        for kg in range(128 // LANES):
            krows = iot + 16 * kg
            base = qv[r, pl.ds(16 * kg, LANES)]

            def drow(d, carry):
                vals = plsc.load_gather(gat.at[b], [krows, carry + d])
                slab[b, d, pl.ds(16 * kg, LANES)] = vals
                return carry

            lax.fori_loop(0, D, drow, base)

    def superblock(t, carry):
        sb = w + NW * t
        idx_copy(sb).start()
        idx_copy(sb).wait()
        compute_pidx()
        gather_copy(0, 0).start()
        gather_copy(1, 1).start()

        def pair(rr, carry2):
            for b in range(2):
                r = rr * 2 + b
                gather_copy(r, b).wait()

                @pl.when(r >= 2)
                def _():
                    out_copy(sb, r - 2, b).wait()

                select_transpose(r, b)
                out_copy(sb, r, b).start()

                @pl.when(r + 2 < 8)
                def _():
                    gather_copy(r + 2, b).start()
            return carry2

        lax.fori_loop(0, 4, pair, 0)
        out_copy(sb, 6, 0).wait()
        out_copy(sb, 7, 1).wait()
        return carry

    lax.fori_loop(0, SB_PER_W, superblock, 0)


@jax.jit
def _embed(x, table):
    tableT = table.T
    xt = x.T
    tail_tp = table[FULL_UNITS * 128:, :].reshape(TAIL_PAIRS, 128)
    tp = _transpose_call(tableT, tail_tp)
    out3 = _gather_call(xt, tp)
    return out3.transpose(2, 0, 1)


def kernel(x, table):
    return _embed(x, table.astype(jnp.float32))


# speedup vs baseline: 1.8254x; 1.8254x over previous
import functools

import jax
import jax.numpy as jnp
from jax import lax
from jax.experimental import pallas as pl
from jax.experimental.pallas import tpu as pltpu
from jax.experimental.pallas import tpu_sc as plsc

VOCAB_ = 1000000
D = 64
NUM_CORES = 2
NUM_SUBCORES = 16
NW = NUM_CORES * NUM_SUBCORES
LANES = 16

FULL_UNITS = VOCAB_ // 128
TAIL_PAIRS = (VOCAB_ - FULL_UNITS * 128) // 2
T_ROUNDS = (FULL_UNITS + NW - 1) // NW // 2 + 2

J = 200
I_ = 4096
JG = J // 8
IB = I_ // 128
SB_PER_W = (JG * IB) // NW


def _mesh():
    return plsc.VectorSubcoreMesh(core_axis_name="c", subcore_axis_name="s")


def _wid():
    return lax.axis_index("s") * NUM_CORES + lax.axis_index("c")


@functools.partial(
    pl.kernel,
    out_type=jax.ShapeDtypeStruct((VOCAB_ // 2, 128), jnp.float32),
    mesh=_mesh(),
    scratch_types=[
        pltpu.VMEM((2, D, 128), jnp.float32),
        pltpu.VMEM((2, D, 128), jnp.float32),
        pltpu.SemaphoreType.DMA((2,)),
        pltpu.SemaphoreType.DMA((2,)),
    ],
    compiler_params=pltpu.CompilerParams(use_tc_tiling_on_sc=True,
                                         needs_layout_passes=False),
)
def _transpose_call(tableT, tail_tp, tp, blk, pck, isem, osem):
    w = _wid()
    iot = lax.iota(jnp.int32, LANES)

    def unit_of(t):
        return w + NW * t

    def in_copy(t, b):
        c0 = pl.multiple_of(unit_of(t) * 128, 128)
        return pltpu.make_async_copy(
            tableT.at[:, pl.ds(c0, 128)], blk.at[b], isem.at[b]
        )

    def out_copy(t, b):
        r0 = pl.multiple_of(unit_of(t) * D, 8)
        return pltpu.make_async_copy(
            pck.at[b], tp.at[pl.ds(r0, D), :], osem.at[b]
        )

    def do_unit(b):
        @plsc.parallel_loop(0, 64, unroll=4)
        def krow(k):
            for half in range(2):
                col = jnp.full((LANES,), 2 * k + half, jnp.int32)
                for g in range(D // LANES):
                    vals = plsc.load_gather(blk.at[b], [iot + 16 * g, col])
                    pck[b, k, pl.ds(64 * half + 16 * g, LANES)] = vals

    for b in range(2):
        @pl.when(unit_of(b) < FULL_UNITS)
        def _():
            in_copy(b, b).start()

    def rounds(tt, carry):
        for b in range(2):
            t = tt * 2 + b

            @pl.when((t >= 2) & (unit_of(t - 2) < FULL_UNITS))
            def _():
                out_copy(t - 2, b).wait()

            @pl.when(unit_of(t) < FULL_UNITS)
            def _():
                in_copy(t, b).wait()
                do_unit(b)
                out_copy(t, b).start()

                @pl.when(unit_of(t + 2) < FULL_UNITS)
                def _():
                    in_copy(t + 2, b).start()
        return carry

    lax.fori_loop(0, T_ROUNDS, rounds, 0)

    @pl.when(w == 0)
    def _():
        pltpu.sync_copy(tail_tp, tp.at[pl.ds(FULL_UNITS * D, TAIL_PAIRS), :])


@functools.partial(
    pl.kernel,
    out_type=jax.ShapeDtypeStruct((J, D, I_), jnp.float32),
    mesh=_mesh(),
    scratch_types=[
        pltpu.VMEM((8, 128), jnp.int32),
        pltpu.VMEM((8, 128), jnp.int32),
        pltpu.VMEM((8, 128), jnp.int32),
        pltpu.VMEM((2, 128, 128), jnp.float32),
        pltpu.VMEM((2, D, 128), jnp.float32),
        pltpu.SemaphoreType.DMA,
        pltpu.SemaphoreType.DMA((2,)),
        pltpu.SemaphoreType.DMA((2,)),
    ],
    compiler_params=pltpu.CompilerParams(use_tc_tiling_on_sc=True,
                                         needs_layout_passes=False),
)
def _gather_call(xt, tp, out3, xv, pv, qv, gat, slab, xsem, gsem, osem):
    w = _wid()
    iot = lax.iota(jnp.int32, LANES)

    def idx_copy(sb):
        jg = sb // IB
        ib = lax.rem(sb, IB)
        return pltpu.make_async_copy(
            xt.at[pl.ds(pl.multiple_of(jg * 8, 8), 8),
                  pl.ds(pl.multiple_of(ib * 128, 128), 128)],
            xv, xsem)

    def gather_copy(r, b):
        return pltpu.make_async_copy(tp.at[pv.at[r]], gat.at[b], gsem.at[b])

    def out_copy(sb, r, b):
        jg = sb // IB
        ib = lax.rem(sb, IB)
        return pltpu.make_async_copy(
            slab.at[b],
            out3.at[jg * 8 + r, :, pl.ds(pl.multiple_of(ib * 128, 128), 128)],
            osem.at[b])

    def compute_pidx():
        def row(r, carry):
            for g in range(128 // LANES):
                v = xv[r, pl.ds(16 * g, LANES)]
                pv[r, pl.ds(16 * g, LANES)] = lax.shift_right_logical(v, 1)
                qv[r, pl.ds(16 * g, LANES)] = lax.shift_left(
                    lax.bitwise_and(v, 1), 6)
            return carry
        lax.fori_loop(0, 8, row, 0)

    def select_transpose(r, b):
        for kg in range(128 // LANES):
            krows = iot + 16 * kg
            base = qv[r, pl.ds(16 * kg, LANES)]

            @plsc.parallel_loop(0, D, unroll=4)
            def drow(d):
                vals = plsc.load_gather(gat.at[b], [krows, base + d])
                slab[b, d, pl.ds(16 * kg, LANES)] = vals

    def superblock(t, carry):
        sb = w + NW * t
        idx_copy(sb).start()
        idx_copy(sb).wait()
        compute_pidx()
        gather_copy(0, 0).start()
        gather_copy(1, 1).start()

        def pair(rr, carry2):
            for b in range(2):
                r = rr * 2 + b
                gather_copy(r, b).wait()

                @pl.when(r >= 2)
                def _():
                    out_copy(sb, r - 2, b).wait()

                select_transpose(r, b)
                out_copy(sb, r, b).start()

                @pl.when(r + 2 < 8)
                def _():
                    gather_copy(r + 2, b).start()
            return carry2

        lax.fori_loop(0, 4, pair, 0)
        out_copy(sb, 6, 0).wait()
        out_copy(sb, 7, 1).wait()
        return carry

    lax.fori_loop(0, SB_PER_W, superblock, 0)


@jax.jit
def _embed(x, table):
    tableT = table.T
    xt = x.T
    tail_tp = table[FULL_UNITS * 128:, :].reshape(TAIL_PAIRS, 128)
    tp = _transpose_call(tableT, tail_tp)
    out3 = _gather_call(xt, tp)
    return out3.transpose(2, 0, 1)


def kernel(x, table):
    return _embed(x, table.astype(jnp.float32))


# speedup vs baseline: 5.1975x; 2.8473x over previous
import functools

import jax
import jax.numpy as jnp
from jax import lax
from jax.experimental import pallas as pl
from jax.experimental.pallas import tpu as pltpu
from jax.experimental.pallas import tpu_sc as plsc

VOCAB_ = 1000000
D = 64
NUM_CORES = 2
NUM_SUBCORES = 16
NW = NUM_CORES * NUM_SUBCORES
LANES = 16

FULL_UNITS = VOCAB_ // 128
TAIL_PAIRS = (VOCAB_ - FULL_UNITS * 128) // 2
T_ROUNDS = (FULL_UNITS + NW - 1) // NW // 2 + 2

J = 200
I_ = 4096
JG = J // 8
IB = I_ // 128
SB_PER_W = (JG * IB) // NW


def _mesh():
    return plsc.VectorSubcoreMesh(core_axis_name="c", subcore_axis_name="s")


def _wid():
    return lax.axis_index("s") * NUM_CORES + lax.axis_index("c")


@functools.partial(
    pl.kernel,
    out_type=jax.ShapeDtypeStruct((VOCAB_ // 2, 128), jnp.float32),
    mesh=_mesh(),
    scratch_types=[
        pltpu.VMEM((2, D, 128), jnp.float32),
        pltpu.VMEM((2, D, 128), jnp.float32),
        pltpu.SemaphoreType.DMA((2,)),
        pltpu.SemaphoreType.DMA((2,)),
    ],
    compiler_params=pltpu.CompilerParams(use_tc_tiling_on_sc=True,
                                         needs_layout_passes=False),
)
def _transpose_call(tableT, tail_tp, tp, blk, pck, isem, osem):
    w = _wid()
    iot = lax.iota(jnp.int32, LANES)

    def unit_of(t):
        return w + NW * t

    def in_copy(t, b):
        c0 = pl.multiple_of(unit_of(t) * 128, 128)
        return pltpu.make_async_copy(
            tableT.at[:, pl.ds(c0, 128)], blk.at[b], isem.at[b]
        )

    def out_copy(t, b):
        r0 = pl.multiple_of(unit_of(t) * D, 8)
        return pltpu.make_async_copy(
            pck.at[b], tp.at[pl.ds(r0, D), :], osem.at[b]
        )

    lrow = [lax.iota(jnp.int32, LANES) + 16 * db for db in range(4)]
    scol = [lax.iota(jnp.int32, LANES) + c0
            for c0 in (0, 16, 32, 48, 64, 80, 96, 112)]

    def do_unit(b):
        @plsc.parallel_loop(0, LANES, unroll=2)
        def diag(i):
            rot = lax.bitwise_and(iot + i, 15)
            rot2 = rot + rot
            for kb in range(4):
                srow = rot + 16 * kb
                for half in range(2):
                    lcol = rot2 + (32 * kb + half)
                    for db in range(4):
                        vals = plsc.load_gather(blk.at[b], [lrow[db], lcol])
                        plsc.store_scatter(
                            pck.at[b], [srow, scol[4 * half + db]], vals)

    for b in range(2):
        @pl.when(unit_of(b) < FULL_UNITS)
        def _():
            in_copy(b, b).start()

    def rounds(tt, carry):
        for b in range(2):
            t = tt * 2 + b

            @pl.when((t >= 2) & (unit_of(t - 2) < FULL_UNITS))
            def _():
                out_copy(t - 2, b).wait()

            @pl.when(unit_of(t) < FULL_UNITS)
            def _():
                in_copy(t, b).wait()
                do_unit(b)
                out_copy(t, b).start()

                @pl.when(unit_of(t + 2) < FULL_UNITS)
                def _():
                    in_copy(t + 2, b).start()
        return carry

    lax.fori_loop(0, T_ROUNDS, rounds, 0)

    @pl.when(w == 0)
    def _():
        pltpu.sync_copy(tail_tp, tp.at[pl.ds(FULL_UNITS * D, TAIL_PAIRS), :])


@functools.partial(
    pl.kernel,
    out_type=jax.ShapeDtypeStruct((J, D, I_), jnp.float32),
    mesh=_mesh(),
    scratch_types=[
        pltpu.VMEM((8, 128), jnp.int32),
        pltpu.VMEM((8, 128), jnp.int32),
        pltpu.VMEM((8, 128), jnp.int32),
        pltpu.VMEM((2, 128, 128), jnp.float32),
        pltpu.VMEM((2, D, 128), jnp.float32),
        pltpu.SemaphoreType.DMA,
        pltpu.SemaphoreType.DMA((2,)),
        pltpu.SemaphoreType.DMA((2,)),
    ],
    compiler_params=pltpu.CompilerParams(use_tc_tiling_on_sc=True,
                                         needs_layout_passes=False),
)
def _gather_call(xt, tp, out3, xv, pv, qv, gat, slab, xsem, gsem, osem):
    w = _wid()
    iot = lax.iota(jnp.int32, LANES)

    def idx_copy(sb):
        jg = sb // IB
        ib = lax.rem(sb, IB)
        return pltpu.make_async_copy(
            xt.at[pl.ds(pl.multiple_of(jg * 8, 8), 8),
                  pl.ds(pl.multiple_of(ib * 128, 128), 128)],
            xv, xsem)

    def gather_copy(r, b):
        return pltpu.make_async_copy(tp.at[pv.at[r]], gat.at[b], gsem.at[b])

    def out_copy(sb, r, b):
        jg = sb // IB
        ib = lax.rem(sb, IB)
        return pltpu.make_async_copy(
            slab.at[b],
            out3.at[jg * 8 + r, :, pl.ds(pl.multiple_of(ib * 128, 128), 128)],
            osem.at[b])

    def compute_pidx():
        def row(r, carry):
            for g in range(128 // LANES):
                v = xv[r, pl.ds(16 * g, LANES)]
                pv[r, pl.ds(16 * g, LANES)] = lax.shift_right_logical(v, 1)
                qv[r, pl.ds(16 * g, LANES)] = lax.shift_left(
                    lax.bitwise_and(v, 1), 6)
            return carry
        lax.fori_loop(0, 8, row, 0)

    scol = [lax.iota(jnp.int32, LANES) + 16 * kb for kb in range(8)]

    def select_transpose(r, b):
        qb = [qv[r, pl.ds(16 * kb, LANES)] for kb in range(8)]

        @plsc.parallel_loop(0, LANES, unroll=2)
        def diag(i):
            rot = lax.bitwise_and(iot + i, 15)
            for dg in range(4):
                rotdg = rot + 16 * dg
                for kb in range(8):
                    vals = plsc.load_gather(
                        gat.at[b], [scol[kb], qb[kb] + rotdg])
                    plsc.store_scatter(slab.at[b], [rotdg, scol[kb]], vals)

    def superblock(t, carry):
        sb = w + NW * t
        idx_copy(sb).start()
        idx_copy(sb).wait()
        compute_pidx()
        gather_copy(0, 0).start()
        gather_copy(1, 1).start()

        def pair(rr, carry2):
            for b in range(2):
                r = rr * 2 + b
                gather_copy(r, b).wait()

                @pl.when(r >= 2)
                def _():
                    out_copy(sb, r - 2, b).wait()

                select_transpose(r, b)
                out_copy(sb, r, b).start()

                @pl.when(r + 2 < 8)
                def _():
                    gather_copy(r + 2, b).start()
            return carry2

        lax.fori_loop(0, 4, pair, 0)
        out_copy(sb, 6, 0).wait()
        out_copy(sb, 7, 1).wait()
        return carry

    lax.fori_loop(0, SB_PER_W, superblock, 0)


@jax.jit
def _embed(x, table):
    tableT = table.T
    xt = x.T
    tail_tp = table[FULL_UNITS * 128:, :].reshape(TAIL_PAIRS, 128)
    tp = _transpose_call(tableT, tail_tp)
    out3 = _gather_call(xt, tp)
    return out3.transpose(2, 0, 1)


def kernel(x, table):
    return _embed(x, table.astype(jnp.float32))
